# baseline (speedup 1.0000x reference)
import functools

import jax
import jax.numpy as jnp
from jax import lax
from jax.experimental import pallas as pl
from jax.experimental.pallas import tpu as pltpu
from jax.experimental.pallas import tpu_sc as plsc

D_MODEL = 64
SEQ = 50
BATCH = 16384

_NC, _NS = 2, 16
_NW = _NC * _NS

_BLK = 128
_NBLK = BATCH // _BLK
_BLK_PER_W = _NBLK // _NW
_W_IDX = _BLK_PER_W * _BLK * SEQ
_UNITS = _BLK_PER_W * SEQ


def _gather_call(table, idx):
    mesh = plsc.VectorSubcoreMesh(core_axis_name="c", subcore_axis_name="s")

    @functools.partial(
        pl.kernel,
        mesh=mesh,
        out_type=jax.ShapeDtypeStruct((SEQ, 8, _NBLK, 8, _BLK), jnp.float32),
        scratch_types=[
            pltpu.VMEM((_W_IDX,), jnp.int32),
            pltpu.VMEM((_BLK,), jnp.int32),
            pltpu.VMEM((_BLK,), jnp.int32),
            pltpu.VMEM((_BLK, D_MODEL), jnp.float32),
            pltpu.VMEM((_BLK, D_MODEL), jnp.float32),
            pltpu.VMEM((8, 1, 8, _BLK), jnp.float32),
            pltpu.VMEM((8, 1, 8, _BLK), jnp.float32),
            pltpu.SemaphoreType.DMA,
            pltpu.SemaphoreType.DMA,
            pltpu.SemaphoreType.DMA,
            pltpu.SemaphoreType.DMA,
            pltpu.SemaphoreType.DMA,
        ],
        compiler_params=pltpu.CompilerParams(
            use_tc_tiling_on_sc=False, needs_layout_passes=False),
    )
    def k(table_hbm, idx_hbm, out_hbm,
          iall, cidx0, cidx1, rows0, rows1, unit0, unit1,
          s_i, sg0, sg1, so0, so1):
        cidx = (cidx0, cidx1)
        rows = (rows0, rows1)
        unit = (unit0, unit1)
        s_g = (sg0, sg1)
        s_o = (so0, so1)

        wid = lax.axis_index("s") * _NC + lax.axis_index("c")

        iota16 = lax.iota(jnp.int32, 16)
        ivec50 = iota16 * SEQ
        row16 = [iota16 + 16 * kk for kk in range(8)]

        def gather_copy(p):
            return pltpu.make_async_copy(table_hbm.at[cidx[p]], rows[p], s_g[p])

        def out_copy(u, p):
            blk = u // SEQ
            s = u % SEQ
            b1 = wid * _BLK_PER_W + blk
            return pltpu.make_async_copy(
                unit[p], out_hbm.at[s, :, pl.ds(b1, 1), :, :], s_o[p])

        def build_cidx(u, p):
            blk = u // SEQ
            s = u % SEQ
            base = blk * (_BLK * SEQ) + s
            for kk in range(8):
                offs = ivec50 + (base + kk * 16 * SEQ)
                cidx[p][pl.ds(16 * kk, 16)] = plsc.load_gather(iall, [offs])

        def transpose(p):
            dvec = jnp.zeros((16,), jnp.int32)
            for d in range(D_MODEL):
                for kk in range(8):
                    v = plsc.load_gather(rows[p], [row16[kk], dvec])
                    unit[p][d // 8, 0, d % 8, pl.ds(16 * kk, 16)] = v
                dvec = dvec + 1

        pltpu.make_async_copy(
            idx_hbm.at[pl.ds(wid * _W_IDX, _W_IDX)], iall, s_i).start()
        pltpu.make_async_copy(
            idx_hbm.at[pl.ds(wid * _W_IDX, _W_IDX)], iall, s_i).wait()

        build_cidx(0, 0)
        gather_copy(0).start()
        build_cidx(1, 1)
        gather_copy(1).start()

        def pair(i, carry):
            for sub in (0, 1):
                u = 2 * i + sub
                gather_copy(sub).wait()

                @pl.when(u >= 2)
                def _():
                    out_copy(u - 2, sub).wait()

                transpose(sub)
                out_copy(u, sub).start()

                @pl.when(u + 2 < _UNITS)
                def _():
                    build_cidx(u + 2, sub)
                    gather_copy(sub).start()

            return carry

        lax.fori_loop(0, _UNITS // 2, pair, 0)
        out_copy(_UNITS - 2, 0).wait()
        out_copy(_UNITS - 1, 1).wait()

    return k(table, idx)


def kernel(data, table):
    idx = data.reshape(-1)
    out5 = _gather_call(table, idx)
    return out5.transpose(2, 4, 0, 1, 3).reshape(BATCH, SEQ, D_MODEL)


# speedup vs baseline: 1.1571x; 1.1571x over previous
import functools

import jax
import jax.numpy as jnp
from jax import lax
from jax.experimental import pallas as pl
from jax.experimental.pallas import tpu as pltpu
from jax.experimental.pallas import tpu_sc as plsc

D_MODEL = 64
SEQ = 50
BATCH = 16384

_NC, _NS = 2, 16
_NW = _NC * _NS

_BLK = 128
_NBLK = BATCH // _BLK
_BLK_PER_W = _NBLK // _NW
_W_IDX = _BLK_PER_W * _BLK * SEQ
_UNITS = _BLK_PER_W * SEQ


def _gather_call(table, idx):
    mesh = plsc.VectorSubcoreMesh(core_axis_name="c", subcore_axis_name="s")

    @functools.partial(
        pl.kernel,
        mesh=mesh,
        out_type=jax.ShapeDtypeStruct((SEQ, 8, _NBLK, 8, _BLK), jnp.float32),
        scratch_types=[
            pltpu.VMEM((_W_IDX,), jnp.int32),
            pltpu.VMEM((_BLK,), jnp.int32),
            pltpu.VMEM((_BLK,), jnp.int32),
            pltpu.VMEM((_BLK, D_MODEL), jnp.float32),
            pltpu.VMEM((_BLK, D_MODEL), jnp.float32),
            pltpu.VMEM((8, 1, 8, _BLK), jnp.float32),
            pltpu.VMEM((8, 1, 8, _BLK), jnp.float32),
            pltpu.SemaphoreType.DMA,
            pltpu.SemaphoreType.DMA,
            pltpu.SemaphoreType.DMA,
            pltpu.SemaphoreType.DMA,
            pltpu.SemaphoreType.DMA,
        ],
        compiler_params=pltpu.CompilerParams(
            use_tc_tiling_on_sc=False, needs_layout_passes=False),
    )
    def k(table_hbm, idx_hbm, out_hbm,
          iall, cidx0, cidx1, rows0, rows1, unit0, unit1,
          s_i, sg0, sg1, so0, so1):
        cidx = (cidx0, cidx1)
        rows = (rows0, rows1)
        unit = (unit0, unit1)
        s_g = (sg0, sg1)
        s_o = (so0, so1)

        wid = lax.axis_index("s") * _NC + lax.axis_index("c")

        iota16 = lax.iota(jnp.int32, 16)
        ivec50 = iota16 * SEQ
        row16 = [iota16 + 16 * kk for kk in range(8)]

        def gather_copy(p):
            return pltpu.make_async_copy(table_hbm.at[cidx[p]], rows[p], s_g[p])

        def out_copy(u, p):
            blk = u // SEQ
            s = u % SEQ
            b1 = wid * _BLK_PER_W + blk
            return pltpu.make_async_copy(
                unit[p], out_hbm.at[s, :, pl.ds(b1, 1), :, :], s_o[p])

        def build_cidx(u, p):
            blk = u // SEQ
            s = u % SEQ
            base = blk * (_BLK * SEQ) + s
            for kk in range(8):
                offs = ivec50 + (base + kk * 16 * SEQ)
                cidx[p][pl.ds(16 * kk, 16)] = plsc.load_gather(iall, [offs])

        def transpose(p):
            dvec = jnp.zeros((16,), jnp.int32)
            for d in range(D_MODEL):
                vs = [plsc.load_gather(rows[p], [row16[kk], dvec])
                      for kk in range(8)]
                for kk in range(8):
                    unit[p][d // 8, 0, d % 8, pl.ds(16 * kk, 16)] = vs[kk]
                dvec = dvec + 1

        pltpu.make_async_copy(
            idx_hbm.at[pl.ds(wid * _W_IDX, _W_IDX)], iall, s_i).start()
        pltpu.make_async_copy(
            idx_hbm.at[pl.ds(wid * _W_IDX, _W_IDX)], iall, s_i).wait()

        build_cidx(0, 0)
        gather_copy(0).start()
        build_cidx(1, 1)
        gather_copy(1).start()

        def pair(i, carry):
            for sub in (0, 1):
                u = 2 * i + sub
                gather_copy(sub).wait()

                @pl.when(u >= 2)
                def _():
                    out_copy(u - 2, sub).wait()

                transpose(sub)
                out_copy(u, sub).start()

                @pl.when(u + 2 < _UNITS)
                def _():
                    build_cidx(u + 2, sub)
                    gather_copy(sub).start()

            return carry

        lax.fori_loop(0, _UNITS // 2, pair, 0)
        out_copy(_UNITS - 2, 0).wait()
        out_copy(_UNITS - 1, 1).wait()

    return k(table, idx)


def kernel(data, table):
    idx = data.reshape(-1)
    out5 = _gather_call(table, idx)
    return out5.transpose(2, 4, 0, 1, 3).reshape(BATCH, SEQ, D_MODEL)


# speedup vs baseline: 2.4784x; 2.1419x over previous
import functools

import jax
import jax.numpy as jnp
from jax import lax
from jax.experimental import pallas as pl
from jax.experimental.pallas import tpu as pltpu
from jax.experimental.pallas import tpu_sc as plsc

D_MODEL = 64
SEQ = 50
BATCH = 16384

_NC, _NS = 2, 16
_NW = _NC * _NS

_BLK = 128
_NBLK = BATCH // _BLK
_BLK_PER_W = _NBLK // _NW
_W_IDX = _BLK_PER_W * _BLK * SEQ
_UNITS = _BLK_PER_W * SEQ


def _gather_call(table, idx):
    mesh = plsc.VectorSubcoreMesh(core_axis_name="c", subcore_axis_name="s")

    @functools.partial(
        pl.kernel,
        mesh=mesh,
        out_type=jax.ShapeDtypeStruct((SEQ, 8, _NBLK, 8, _BLK), jnp.float32),
        scratch_types=[
            pltpu.VMEM((_W_IDX,), jnp.int32),
            pltpu.VMEM((_BLK,), jnp.int32),
            pltpu.VMEM((_BLK,), jnp.int32),
            pltpu.VMEM((_BLK, D_MODEL), jnp.float32),
            pltpu.VMEM((_BLK, D_MODEL), jnp.float32),
            pltpu.VMEM((8, 1, 8, _BLK), jnp.float32),
            pltpu.VMEM((8, 1, 8, _BLK), jnp.float32),
            pltpu.SemaphoreType.DMA,
            pltpu.SemaphoreType.DMA,
            pltpu.SemaphoreType.DMA,
            pltpu.SemaphoreType.DMA,
            pltpu.SemaphoreType.DMA,
        ],
        compiler_params=pltpu.CompilerParams(
            use_tc_tiling_on_sc=False, needs_layout_passes=False),
    )
    def k(table_hbm, idx_hbm, out_hbm,
          iall, cidx0, cidx1, rows0, rows1, unit0, unit1,
          s_i, sg0, sg1, so0, so1):
        cidx = (cidx0, cidx1)
        rows = (rows0, rows1)
        unit = (unit0, unit1)
        s_g = (sg0, sg1)
        s_o = (so0, so1)

        wid = lax.axis_index("s") * _NC + lax.axis_index("c")

        iota16 = lax.iota(jnp.int32, 16)
        ivec50 = iota16 * SEQ
        row16 = [iota16 + 16 * kk for kk in range(8)]

        def gather_copy(p):
            return pltpu.make_async_copy(table_hbm.at[cidx[p]], rows[p], s_g[p])

        def out_copy(u, p):
            blk = u // SEQ
            s = u % SEQ
            b1 = wid * _BLK_PER_W + blk
            return pltpu.make_async_copy(
                unit[p], out_hbm.at[s, :, pl.ds(b1, 1), :, :], s_o[p])

        def build_cidx(u, p):
            blk = u // SEQ
            s = u % SEQ
            base = blk * (_BLK * SEQ) + s
            for kk in range(8):
                offs = ivec50 + (base + kk * 16 * SEQ)
                cidx[p][pl.ds(16 * kk, 16)] = plsc.load_gather(iall, [offs])

        def transpose(p):
            dvec = jnp.zeros((16,), jnp.int32)
            for d in range(D_MODEL):
                vs = [plsc.load_gather(rows[p], [row16[kk], dvec])
                      for kk in range(8)]
                for kk in range(8):
                    unit[p][d // 8, 0, d % 8, pl.ds(16 * kk, 16)] = vs[kk]
                dvec = dvec + 1

        pltpu.make_async_copy(
            idx_hbm.at[pl.ds(wid * _W_IDX, _W_IDX)], iall, s_i).start()
        pltpu.make_async_copy(
            idx_hbm.at[pl.ds(wid * _W_IDX, _W_IDX)], iall, s_i).wait()

        build_cidx(0, 0)
        gather_copy(0).start()
        build_cidx(1, 1)
        gather_copy(1).start()

        def pair(i, carry):
            for sub in (0, 1):
                u = 2 * i + sub
                gather_copy(sub).wait()

                @pl.when(u >= 2)
                def _():
                    out_copy(u - 2, sub).wait()

                out_copy(u, sub).start()

                @pl.when(u + 2 < _UNITS)
                def _():
                    build_cidx(u + 2, sub)
                    gather_copy(sub).start()

            return carry

        lax.fori_loop(0, _UNITS // 2, pair, 0)
        out_copy(_UNITS - 2, 0).wait()
        out_copy(_UNITS - 1, 1).wait()

    return k(table, idx)


def kernel(data, table):
    idx = data.reshape(-1)
    out5 = _gather_call(table, idx)
    return out5.transpose(2, 4, 0, 1, 3).reshape(BATCH, SEQ, D_MODEL)
